# Initial kernel scaffold; baseline (speedup 1.0000x reference)
import jax
import jax.numpy as jnp
from jax.experimental import pallas as pl


def kernel(inputs, adj, mask_item, item, embedding, a0, a1, a2, a3, g_w1, g_w2, g_w3, num_w, adj_all):
    raise NotImplementedError("write your pallas kernel here")


# speedup vs baseline: 2.0045x; 2.0045x over previous
import jax
import jax.numpy as jnp
from jax import lax
from jax.experimental import pallas as pl
from jax.experimental.pallas import tpu as pltpu
from jax.experimental.pallas import tpu_sc as plsc

NUM_NODE = 100000
DIM = 128
B = 1024
L = 20
S = 12
LRELU = 0.2

NC = 2
NS = 16
NWORK = NC * NS

NIDS = B * L
NNBR = NIDS * S
NALL = NNBR + 2 * NIDS

IPW = NIDS // NWORK
ICH = 128
INCH = IPW // ICH

RPW = NALL // NWORK
CH = 448
NCHUNK = RPW // CH

BBLK = 8


def _leaky(x, slope=LRELU):
    return jnp.where(x >= 0, x, slope * x)


def _hop_gather_body(adj_hbm, numw_hbm, ids_hbm, nbrs_out, nw_out,
                     idx_v, nbr_buf, nw_buf, gsem):
    wid = lax.axis_index("s") * NC + lax.axis_index("c")
    base = wid * IPW
    pltpu.sync_copy(ids_hbm.at[pl.ds(base, IPW)], idx_v)
    descs = []
    for c in range(INCH):
        sl = pl.ds(c * ICH, ICH)
        descs.append(pltpu.async_copy(adj_hbm.at[idx_v.at[sl]],
                                      nbr_buf.at[sl], gsem))
        descs.append(pltpu.async_copy(numw_hbm.at[idx_v.at[sl]],
                                      nw_buf.at[sl], gsem))
    for d in descs:
        d.wait()
    pltpu.sync_copy(nbr_buf, nbrs_out.at[pl.ds(base, IPW)])
    pltpu.sync_copy(nw_buf, nw_out.at[pl.ds(base, IPW)])


def _row_gather_body(emb_hbm, ids_hbm, rows_out, idx_v, buf0, buf1,
                     gsem, wsem):
    bufs = (buf0, buf1)
    wid = lax.axis_index("s") * NC + lax.axis_index("c")
    base = wid * RPW
    pltpu.sync_copy(ids_hbm.at[pl.ds(wid * NCHUNK, NCHUNK)], idx_v)

    def fire_gather(c, buf):
        return pltpu.async_copy(emb_hbm.at[idx_v.at[c]], buf, gsem)

    def fire_wb(c, buf):
        off = pl.multiple_of(base + c * CH, 8)
        return pltpu.async_copy(buf, rows_out.at[pl.ds(off, CH)], wsem)

    ds_g = [None, None]
    ds_w = [None, None]
    ds_g[0] = fire_gather(0, bufs[0])
    for c in range(NCHUNK):
        b = c & 1
        if c + 1 < NCHUNK:
            nb = (c + 1) & 1
            if ds_w[nb] is not None:
                ds_w[nb].wait()
            ds_g[nb] = fire_gather(c + 1, bufs[nb])
        ds_g[b].wait()
        ds_w[b] = fire_wb(c, bufs[b])
    ds_w[0].wait()
    ds_w[1].wait()


def _tc_body(nv_ref, h_ref, it_ref, adj_ref, mask_ref, nw_ref,
             a4_ref, w1a_ref, w1b_ref, w2_ref, w3a_ref, w3b_ref,
             out_ref, sg_ref):
    h2 = h_ref[...]
    h3 = h2.reshape(BBLK, L, DIM)
    it3 = it_ref[...].reshape(BBLK, L, DIM)
    nv2 = nv_ref[...]
    maskf = mask_ref[...]

    s_sum = jnp.sum(it3 * maskf[:, :, None], axis=1)
    denom = jnp.sum(maskf, axis=1, keepdims=True)
    s_emb = s_sum / denom

    nv3 = nv2.reshape(BBLK, L * S, DIM)
    q2 = (nv3 * s_emb[:, None, :]).reshape(BBLK * L * S, DIM)
    al = jnp.dot(q2, w1a_ref[...])
    al3 = al.reshape(BBLK * L, S, DIM)
    nwv = nw_ref[...].reshape(BBLK * L, S)
    al3 = _leaky(al3 + nwv[:, :, None] * w1b_ref[...][0][None, None, :])
    m = jnp.sum(al3 * w2_ref[...][0][None, None, :], axis=-1)
    m = m - jnp.max(m, axis=-1, keepdims=True)
    e = jnp.exp(m)
    att_g = e / jnp.sum(e, axis=-1, keepdims=True)
    nv4 = nv2.reshape(BBLK * L, S, DIM)
    agg = jnp.sum(nv4 * att_g[:, :, None], axis=1)
    sg2 = jnp.maximum(jnp.dot(h2, w3a_ref[...]) + jnp.dot(agg, w3b_ref[...]), 0.0)
    sg3 = sg2.reshape(BBLK, L, DIM)
    sg_ref[...] = sg3

    a4 = a4_ref[...]
    locs = []
    for i in range(BBLK):
        hi = h3[i]
        hm = (a4[:, None, :] * hi[None, :, :]).reshape(4 * L, DIM)
        ee = _leaky(lax.dot_general(hm, hi, (((1,), (1,)), ((), ()))))
        adji = adj_ref[i]
        att = jnp.full((L, L), -9e15, jnp.float32)
        for r in range(4):
            att = jnp.where(adji == r + 1, ee[r * L:(r + 1) * L, :], att)
        att = att - jnp.max(att, axis=-1, keepdims=True)
        ea = jnp.exp(att)
        att = ea / jnp.sum(ea, axis=-1, keepdims=True)
        locs.append((att @ hi)[None])
    h_local = jnp.concatenate(locs, axis=0)
    out_ref[...] = h_local + sg3


def _sc_hop_gather(adj_all, num_w, ids):
    f = pl.kernel(
        _hop_gather_body,
        out_type=(
            jax.ShapeDtypeStruct((NIDS, S), jnp.int32),
            jax.ShapeDtypeStruct((NIDS, S), jnp.float32),
        ),
        mesh=plsc.VectorSubcoreMesh(core_axis_name="c", subcore_axis_name="s"),
        scratch_types=[
            pltpu.VMEM((IPW,), jnp.int32),
            pltpu.VMEM((IPW, S), jnp.int32),
            pltpu.VMEM((IPW, S), jnp.float32),
            pltpu.SemaphoreType.DMA,
        ],
        compiler_params=pltpu.CompilerParams(use_tc_tiling_on_sc=False),
    )
    return f(adj_all, num_w, ids)


def _sc_row_gather(embedding, ids_all):
    ids2d = ids_all.reshape(NWORK * NCHUNK, CH)
    f = pl.kernel(
        _row_gather_body,
        out_type=jax.ShapeDtypeStruct((NALL, DIM), jnp.float32),
        mesh=plsc.VectorSubcoreMesh(core_axis_name="c", subcore_axis_name="s"),
        scratch_types=[
            pltpu.VMEM((NCHUNK, CH), jnp.int32),
            pltpu.VMEM((CH, DIM), jnp.float32),
            pltpu.VMEM((CH, DIM), jnp.float32),
            pltpu.SemaphoreType.DMA,
            pltpu.SemaphoreType.DMA,
        ],
        compiler_params=pltpu.CompilerParams(use_tc_tiling_on_sc=False),
    )
    return f(embedding, ids2d)


def _tc_compute(rows, adj, maskf, nw3, a4, w1a, w1b, w2r, w3a, w3b):
    nblk = B // BBLK
    h_off = NNBR // (BBLK * L)
    it_off = h_off + B // BBLK
    grid_spec = pl.GridSpec(
        grid=(nblk,),
        in_specs=[
            pl.BlockSpec((BBLK * L * S, DIM), lambda i: (i, 0)),
            pl.BlockSpec((BBLK * L, DIM), lambda i: (i + h_off, 0)),
            pl.BlockSpec((BBLK * L, DIM), lambda i: (i + it_off, 0)),
            pl.BlockSpec((BBLK, L, L), lambda i: (i, 0, 0)),
            pl.BlockSpec((BBLK, L), lambda i: (i, 0)),
            pl.BlockSpec((BBLK, L, S), lambda i: (i, 0, 0)),
            pl.BlockSpec((4, DIM), lambda i: (0, 0)),
            pl.BlockSpec((DIM, DIM), lambda i: (0, 0)),
            pl.BlockSpec((1, DIM), lambda i: (0, 0)),
            pl.BlockSpec((1, DIM), lambda i: (0, 0)),
            pl.BlockSpec((DIM, DIM), lambda i: (0, 0)),
            pl.BlockSpec((DIM, DIM), lambda i: (0, 0)),
        ],
        out_specs=[
            pl.BlockSpec((BBLK, L, DIM), lambda i: (i, 0, 0)),
            pl.BlockSpec((BBLK, L, DIM), lambda i: (i, 0, 0)),
        ],
    )
    return pl.pallas_call(
        _tc_body,
        grid_spec=grid_spec,
        out_shape=[
            jax.ShapeDtypeStruct((B, L, DIM), jnp.float32),
            jax.ShapeDtypeStruct((B, L, DIM), jnp.float32),
        ],
    )(rows, rows, rows, adj, maskf, nw3, a4, w1a, w1b, w2r, w3a, w3b)


def kernel(inputs, adj, mask_item, item, embedding, a0, a1, a2, a3,
           g_w1, g_w2, g_w3, num_w, adj_all):
    ids = inputs.reshape(-1).astype(jnp.int32)
    item_ids = item.reshape(-1).astype(jnp.int32)

    nbrs = jnp.take(adj_all.astype(jnp.int32), ids, axis=0)
    nw = jnp.take(num_w, ids, axis=0)
    ids_all = jnp.concatenate([nbrs.reshape(-1), ids, item_ids])
    rows = _sc_row_gather(embedding, ids_all)

    maskf = mask_item.astype(jnp.float32)
    nw3 = nw.reshape(B, L, S)
    a4 = jnp.concatenate([a0, a1, a2, a3], axis=1).T
    w1a = g_w1[:DIM]
    w1b = g_w1[DIM:DIM + 1]
    w2r = g_w2.T
    w3a = g_w3[:DIM]
    w3b = g_w3[DIM:]

    output, s_global = _tc_compute(rows, adj.astype(jnp.int32), maskf, nw3,
                                   a4, w1a, w1b, w2r, w3a, w3b)
    return (output, s_global)
